# baseline (speedup 1.0000x reference)
import functools

import jax
import jax.numpy as jnp
from jax import lax
from jax.experimental import pallas as pl
from jax.experimental.pallas import tpu as pltpu
from jax.experimental.pallas import tpu_sc as plsc

_B = 8
_NA = 18
_C = 86
_G = 64
_GG = _G * _G
_NSLAB = _B * _NA
_P = 256
_NCHUNK = _GG // _P
_TOTAL = _NSLAB * _NCHUNK
_NW = 32
_PER_W = _TOTAL // _NW
_SXY = 1.05
_HALF = (_SXY - 1.0) / 2.0
_STRIDE = 8.0

_AW = (12.0, 19.0, 40.0)
_AH = (16.0, 36.0, 28.0)
_ANGLES = (-1.0472, -0.5236, 0.0, 0.5236, 1.0472, 1.5708)


def _sigmoid(x):
    return 1.0 / (1.0 + jnp.exp(-x))


def _scalar_select(idx, values):
    out = jnp.float32(values[-1])
    for i in range(len(values) - 2, -1, -1):
        out = jnp.where(idx == i, jnp.float32(values[i]), out)
    return out


def _sc_body(in_hbm, out_hbm, in_v, out_v):
    w = lax.axis_index("s") * 2 + lax.axis_index("c")
    iota = lax.iota(jnp.int32, 16)

    def chunk(k, carry):
        t = w * _PER_W + k
        slab = t // _NCHUNK
        pc = t - slab * _NCHUNK
        a = slab % _NA
        ai = a // 6
        aj = a - ai * 6
        aw = _scalar_select(ai, _AW)
        ah = _scalar_select(ai, _AH)
        aa = _scalar_select(aj, _ANGLES)

        pltpu.sync_copy(in_hbm.at[slab, :, pl.ds(pc * _P, _P)], in_v)

        def box_group(g, c2):
            p_idx = (iota + g * 16) * _C
            gx = ((g % 4) * 16 + iota).astype(jnp.float32)
            gy = (pc * 4 + g // 4).astype(jnp.float32)
            x0 = in_v[0, pl.ds(g * 16, 16)]
            y0 = (_sigmoid(x0) * _SXY - _HALF + gx) * _STRIDE
            plsc.store_scatter(out_v, [p_idx], y0)
            x1 = in_v[1, pl.ds(g * 16, 16)]
            y1 = (_sigmoid(x1) * _SXY - _HALF + gy) * _STRIDE
            plsc.store_scatter(out_v, [p_idx + 1], y1)
            x2 = in_v[2, pl.ds(g * 16, 16)]
            plsc.store_scatter(out_v, [p_idx + 2], jnp.exp(x2) * aw)
            x3 = in_v[3, pl.ds(g * 16, 16)]
            plsc.store_scatter(out_v, [p_idx + 3], jnp.exp(x3) * ah)
            x4 = in_v[4, pl.ds(g * 16, 16)]
            plsc.store_scatter(out_v, [p_idx + 4], x4 + aa)
            return c2

        lax.fori_loop(0, 16, box_group, 0)

        def sig_group(t2, c2):
            c = 5 + t2 // 16
            g = t2 - (t2 // 16) * 16
            x = in_v[c, pl.ds(g * 16, 16)]
            plsc.store_scatter(
                out_v, [(iota + g * 16) * _C + c], _sigmoid(x)
            )
            return c2

        lax.fori_loop(0, (_C - 5) * 16, sig_group, 0)

        pltpu.sync_copy(out_v, out_hbm.at[pl.ds(t * _P * _C, _P * _C)])
        return carry

    lax.fori_loop(0, _PER_W, chunk, 0)


def kernel(output):
    x = output.reshape(_NSLAB, _C, _GG)
    mesh = plsc.VectorSubcoreMesh(core_axis_name="c", subcore_axis_name="s")
    run = functools.partial(
        pl.kernel,
        mesh=mesh,
        out_type=jax.ShapeDtypeStruct((_NSLAB * _GG * _C,), jnp.float32),
        scratch_types=[
            pltpu.VMEM((_C, _P), jnp.float32),
            pltpu.VMEM((_P * _C,), jnp.float32),
        ],
        compiler_params=pltpu.CompilerParams(needs_layout_passes=False),
    )(_sc_body)
    out = run(x)
    return out.reshape(_B, _NA * _GG, _C)


# speedup vs baseline: 1.8832x; 1.8832x over previous
import functools

import jax
import jax.numpy as jnp
from jax import lax
from jax.experimental import pallas as pl
from jax.experimental.pallas import tpu as pltpu
from jax.experimental.pallas import tpu_sc as plsc

_B = 8
_NA = 18
_C = 86
_G = 64
_GG = _G * _G
_NSLAB = _B * _NA
_P = 256
_NCHUNK = _GG // _P
_TOTAL = _NSLAB * _NCHUNK
_NW = 32
_PER_W = _TOTAL // _NW
_SXY = 1.05
_HALF = (_SXY - 1.0) / 2.0
_STRIDE = 8.0

_AW = (12.0, 19.0, 40.0)
_AH = (16.0, 36.0, 28.0)
_ANGLES = (-1.0472, -0.5236, 0.0, 0.5236, 1.0472, 1.5708)


def _sigmoid(x):
    return 1.0 / (1.0 + jnp.exp(-x))


def _scalar_select(idx, values):
    out = jnp.float32(values[-1])
    for i in range(len(values) - 2, -1, -1):
        out = jnp.where(idx == i, jnp.float32(values[i]), out)
    return out


def _sc_body(in_hbm, out_hbm, in_v, out_v):
    w = lax.axis_index("s") * 2 + lax.axis_index("c")
    iota = lax.iota(jnp.int32, 16)

    def chunk(k, carry):
        t = w * _PER_W + k
        slab = t // _NCHUNK
        pc = t - slab * _NCHUNK
        a = slab % _NA
        ai = a // 6
        aj = a - ai * 6
        aw = _scalar_select(ai, _AW)
        ah = _scalar_select(ai, _AH)
        aa = _scalar_select(aj, _ANGLES)

        pltpu.sync_copy(in_hbm.at[slab, :, pl.ds(pc * _P, _P)], in_v)

        gyb = (pc * 4).astype(jnp.float32)
        fiota = iota.astype(jnp.float32)

        @plsc.parallel_loop(0, 16, unroll=4)
        def box_group(g):
            p_idx = iota * _C + g * (16 * _C)
            gx = (lax.rem(g, 4) * 16).astype(jnp.float32) + fiota
            gy = gyb + (g // 4).astype(jnp.float32)
            x0 = in_v[0, pl.ds(g * 16, 16)]
            y0 = (_sigmoid(x0) * _SXY - _HALF + gx) * _STRIDE
            plsc.store_scatter(out_v, [p_idx], y0)
            x1 = in_v[1, pl.ds(g * 16, 16)]
            y1 = (_sigmoid(x1) * _SXY - _HALF + gy) * _STRIDE
            plsc.store_scatter(out_v, [p_idx + 1], y1)
            x2 = in_v[2, pl.ds(g * 16, 16)]
            plsc.store_scatter(out_v, [p_idx + 2], jnp.exp(x2) * aw)
            x3 = in_v[3, pl.ds(g * 16, 16)]
            plsc.store_scatter(out_v, [p_idx + 3], jnp.exp(x3) * ah)
            x4 = in_v[4, pl.ds(g * 16, 16)]
            plsc.store_scatter(out_v, [p_idx + 4], x4 + aa)

        @plsc.parallel_loop(5, _C, unroll=2)
        def sig_row(c):
            base = iota * _C + c
            for g in range(16):
                x = in_v[c, pl.ds(g * 16, 16)]
                plsc.store_scatter(out_v, [base + g * (16 * _C)], _sigmoid(x))

        pltpu.sync_copy(out_v, out_hbm.at[pl.ds(t * _P * _C, _P * _C)])
        return carry

    lax.fori_loop(0, _PER_W, chunk, 0)


def kernel(output):
    x = output.reshape(_NSLAB, _C, _GG)
    mesh = plsc.VectorSubcoreMesh(core_axis_name="c", subcore_axis_name="s")
    run = functools.partial(
        pl.kernel,
        mesh=mesh,
        out_type=jax.ShapeDtypeStruct((_NSLAB * _GG * _C,), jnp.float32),
        scratch_types=[
            pltpu.VMEM((_C, _P), jnp.float32),
            pltpu.VMEM((_P * _C,), jnp.float32),
        ],
        compiler_params=pltpu.CompilerParams(needs_layout_passes=False),
    )(_sc_body)
    out = run(x)
    return out.reshape(_B, _NA * _GG, _C)
